# baseline (speedup 1.0000x reference)
import functools

import jax
import jax.numpy as jnp
from jax import lax
from jax.experimental import pallas as pl
from jax.experimental.pallas import tpu as pltpu
from jax.experimental.pallas import tpu_sc as plsc

F_CAT = 26
F_NUM = 13
D = 64
EPS = 1e-5


def _make_sc_gather(B: int):
    info = plsc.get_sparse_core_info()
    nc, ns = info.num_cores, info.num_subcores
    nw = nc * ns
    assert B % nw == 0
    CHB = B // nw
    assert CHB % 16 == 0
    NBUF = 2

    mesh = plsc.VectorSubcoreMesh(core_axis_name="c", subcore_axis_name="s")

    @functools.partial(
        pl.kernel,
        out_type=jax.ShapeDtypeStruct((B * F_CAT, D), jnp.float32),
        mesh=mesh,
        scratch_types=[
            pltpu.VMEM((NBUF, CHB), jnp.int32),
            pltpu.VMEM((NBUF, CHB), jnp.int32),
            pltpu.VMEM((NBUF, CHB, D), jnp.float32),
        ]
        + [pltpu.SemaphoreType.DMA] * (2 * NBUF),
        compiler_params=pltpu.CompilerParams(use_tc_tiling_on_sc=False),
    )
    def sc_gather(table_hbm, idxt_hbm, out_hbm, idx_v, oidx_v, rows_v, *sems):
        gsem = sems[:NBUF]
        osem = sems[NBUF:]
        wid = lax.axis_index("s") * nc + lax.axis_index("c")
        b0 = wid * CHB

        def fill_oidx(b, f):
            for j in range(CHB // 16):
                v = (b0 + j * 16 + lax.iota(jnp.int32, 16)) * F_CAT + f
                oidx_v[b, pl.ds(j * 16, 16)] = v

        def start_gather(b, f):
            pltpu.sync_copy(idxt_hbm.at[f, pl.ds(b0, CHB)], idx_v.at[b])
            pltpu.async_copy(
                table_hbm.at[f].at[idx_v.at[b]], rows_v.at[b], gsem[b]
            )

        def wait_gather(b, f):
            pltpu.make_async_copy(
                table_hbm.at[f].at[idx_v.at[b]], rows_v.at[b], gsem[b]
            ).wait()

        def start_scatter(b, f):
            fill_oidx(b, f)
            pltpu.async_copy(
                rows_v.at[b], out_hbm.at[oidx_v.at[b]], osem[b]
            )

        def wait_scatter(b):
            pltpu.make_async_copy(
                rows_v.at[b], out_hbm.at[oidx_v.at[b]], osem[b]
            ).wait()

        start_gather(0, 0)
        start_gather(1, 1)
        for f in range(F_CAT):
            b = f % NBUF
            wait_gather(b, f)
            start_scatter(b, f)
            if f + NBUF < F_CAT:
                wait_scatter(b)
                start_gather(b, f + NBUF)
        for b in range(NBUF):
            wait_scatter(b)

    return sc_gather


def _epilogue_body(cat_ref, xn_ref, w_ref, b_ref, g_ref, bt_ref, out_ref):
    cat = cat_ref[...]
    xn = xn_ref[...]
    num = xn[:, :, None] * w_ref[...][None] + b_ref[...][None]
    x = jnp.concatenate([cat, num], axis=1)
    mu = jnp.mean(x, axis=-1, keepdims=True)
    xc = x - mu
    var = jnp.mean(xc * xc, axis=-1, keepdims=True)
    y = xc * lax.rsqrt(var + EPS)
    out_ref[...] = y * g_ref[...][None] + bt_ref[...][None]


def _epilogue(cat, x_num, W_num, b_num, gamma, beta):
    B = cat.shape[0]
    BT = 256
    grid = (B // BT,)
    g2 = gamma.reshape(1, D)
    bt2 = beta.reshape(1, D)
    return pl.pallas_call(
        _epilogue_body,
        grid=grid,
        in_specs=[
            pl.BlockSpec((BT, F_CAT, D), lambda i: (i, 0, 0)),
            pl.BlockSpec((BT, F_NUM), lambda i: (i, 0)),
            pl.BlockSpec((F_NUM, D), lambda i: (0, 0)),
            pl.BlockSpec((F_NUM, D), lambda i: (0, 0)),
            pl.BlockSpec((1, D), lambda i: (0, 0)),
            pl.BlockSpec((1, D), lambda i: (0, 0)),
        ],
        out_specs=pl.BlockSpec((BT, F_CAT + F_NUM, D), lambda i: (i, 0, 0)),
        out_shape=jax.ShapeDtypeStruct((B, F_CAT + F_NUM, D), jnp.float32),
        compiler_params=pltpu.CompilerParams(
            dimension_semantics=("parallel",)
        ),
    )(cat, x_num, W_num, b_num, g2, bt2)


def kernel(x_cat, x_num, tables, W_num, b_num, gamma, beta):
    B = x_cat.shape[0]
    idx_t = x_cat.astype(jnp.int32).T
    cat_flat = _make_sc_gather(B)(tables, idx_t)
    cat = cat_flat.reshape(B, F_CAT, D)
    return _epilogue(cat, x_num, W_num, b_num, gamma, beta)


# speedup vs baseline: 20.3474x; 20.3474x over previous
import functools

import jax
import jax.numpy as jnp
from jax import lax
from jax.experimental import pallas as pl
from jax.experimental.pallas import tpu as pltpu
from jax.experimental.pallas import tpu_sc as plsc

F_CAT = 26
F_NUM = 13
F_TOT = F_CAT + F_NUM
D = 64
EPS = 1e-5


def _make_sc_gather(B: int, V: int):
    info = plsc.get_sparse_core_info()
    nc, ns = info.num_cores, info.num_subcores
    nw = nc * ns
    nrow = F_CAT * D
    assert nrow % nw == 0
    rpw = nrow // nw

    mesh = plsc.VectorSubcoreMesh(core_axis_name="c", subcore_axis_name="s")

    @functools.partial(
        pl.kernel,
        out_type=jax.ShapeDtypeStruct((nrow, B), jnp.float32),
        mesh=mesh,
        scratch_types=[
            pltpu.VMEM((V,), jnp.float32),
            pltpu.VMEM((B,), jnp.int32),
            pltpu.VMEM((B,), jnp.float32),
        ],
        compiler_params=pltpu.CompilerParams(
            use_tc_tiling_on_sc=True, needs_layout_passes=False
        ),
    )
    def sc_gather(a2_hbm, idxt_hbm, out_hbm, row_v, idx_v, val_v):
        wid = lax.axis_index("s") * nc + lax.axis_index("c")
        r0 = wid * rpw

        @pl.loop(0, rpw)
        def _(k):
            r = r0 + k
            f = r // D
            pltpu.sync_copy(idxt_hbm.at[pl.ds(f * B, B)], idx_v)
            pltpu.sync_copy(a2_hbm.at[r], row_v)
            for j in range(B // 16):
                iv = idx_v[pl.ds(j * 16, 16)]
                val_v[pl.ds(j * 16, 16)] = plsc.load_gather(row_v, [iv])
            pltpu.sync_copy(val_v, out_hbm.at[r])

    return sc_gather


def _epilogue_body(cat_ref, xnt_ref, w_ref, b_ref, g_ref, bt_ref, out_ref):
    cat = cat_ref[...].reshape(F_CAT, D, cat_ref.shape[1])
    xn = xnt_ref[...]
    w = w_ref[...]
    b = b_ref[...]
    num = xn[:, None, :] * w[:, :, None] + b[:, :, None]
    x = jnp.concatenate([cat, num], axis=0)
    mu = jnp.mean(x, axis=1, keepdims=True)
    xc = x - mu
    var = jnp.mean(xc * xc, axis=1, keepdims=True)
    y = xc * lax.rsqrt(var + EPS)
    g = g_ref[...][None, :, :]
    bt = bt_ref[...][None, :, :]
    out_ref[...] = y * g + bt


def _epilogue(cat_t, x_num_t, W_num, b_num, gamma, beta):
    B = cat_t.shape[1]
    BT = 512
    grid = (B // BT,)
    g2 = gamma.reshape(D, 1)
    bt2 = beta.reshape(D, 1)
    return pl.pallas_call(
        _epilogue_body,
        grid=grid,
        in_specs=[
            pl.BlockSpec((F_CAT * D, BT), lambda i: (0, i)),
            pl.BlockSpec((F_NUM, BT), lambda i: (0, i)),
            pl.BlockSpec((F_NUM, D), lambda i: (0, 0)),
            pl.BlockSpec((F_NUM, D), lambda i: (0, 0)),
            pl.BlockSpec((D, 1), lambda i: (0, 0)),
            pl.BlockSpec((D, 1), lambda i: (0, 0)),
        ],
        out_specs=pl.BlockSpec((F_TOT, D, BT), lambda i: (0, 0, i)),
        out_shape=jax.ShapeDtypeStruct((F_TOT, D, B), jnp.float32),
        compiler_params=pltpu.CompilerParams(
            dimension_semantics=("parallel",)
        ),
    )(cat_t, x_num_t, W_num, b_num, g2, bt2)


def kernel(x_cat, x_num, tables, W_num, b_num, gamma, beta):
    B = x_cat.shape[0]
    V = tables.shape[1]
    a2 = jnp.swapaxes(tables, 1, 2).reshape(F_CAT * D, V)
    idxt = x_cat.astype(jnp.int32).T.reshape(F_CAT * B)
    cat_t = _make_sc_gather(B, V)(a2, idxt)
    out_t = _epilogue(cat_t, x_num.T, W_num, b_num, gamma, beta)
    return out_t.transpose(2, 0, 1)
